# Initial kernel scaffold; baseline (speedup 1.0000x reference)
import jax
import jax.numpy as jnp
from jax.experimental import pallas as pl


def kernel(x, edge_index, W1, a_src1, a_dst1, b1, W2, a_src2, a_dst2, b2):
    raise NotImplementedError("write your pallas kernel here")


# speedup vs baseline: 4.7817x; 4.7817x over previous
import functools

import jax
import jax.numpy as jnp
from jax.experimental import pallas as pl

_H1 = 8
_C1 = 8


def _cdiv(a, b):
    return (a + b - 1) // b


def _node1_body(x_ref, w_ref, asrc_ref, adst_ref, h_ref, s_ref, d_ref):
    h = jnp.dot(x_ref[...], w_ref[...], preferred_element_type=jnp.float32)
    h_ref[...] = h
    for hh in range(_H1):
        blk = h[:, hh * _C1:(hh + 1) * _C1]
        s_ref[:, hh:hh + 1] = jnp.sum(blk * asrc_ref[hh:hh + 1, :], axis=-1,
                                      keepdims=True)
        d_ref[:, hh:hh + 1] = jnp.sum(blk * adst_ref[hh:hh + 1, :], axis=-1,
                                      keepdims=True)


def _edge1_body(ae_s_ref, ae_d_ref, hs_ref, p_ref, msg_ref):
    a = ae_s_ref[...] + ae_d_ref[...]
    a = jnp.where(a >= 0, a, 0.2 * a)
    p = jnp.exp(a)
    p_ref[...] = p
    for hh in range(_H1):
        msg_ref[:, hh * _C1:(hh + 1) * _C1] = (
            hs_ref[:, hh * _C1:(hh + 1) * _C1] * p[:, hh:hh + 1])


def _node2_body(out1_ref, den_ref, b1_ref, w2_ref, as2_ref, ad2_ref,
                z_ref, zs_ref, zd_ref):
    den = den_ref[...]
    cols = []
    for hh in range(_H1):
        cols.append(out1_ref[:, hh * _C1:(hh + 1) * _C1] /
                    (den[:, hh:hh + 1] + 1e-16))
    h2 = jnp.concatenate(cols, axis=1) + b1_ref[...]
    h2 = jnp.where(h2 > 0, h2, jnp.exp(h2) - 1.0)
    z = jnp.dot(h2, w2_ref[...], preferred_element_type=jnp.float32)
    z_ref[...] = z
    zs_ref[...] = z * as2_ref[...]
    zd_ref[...] = z * ad2_ref[...]


def _edge2_body(ae_ref, zg_ref, p_ref, m_ref):
    a = ae_ref[...]
    a = jnp.where(a >= 0, a, 0.2 * a)
    p = jnp.exp(a)
    p_ref[...] = p
    m_ref[...] = p * zg_ref[...]


def _final_body(num_ref, den_ref, b2_ref, o_ref):
    v = num_ref[...] / (den_ref[...] + 1e-16) + b2_ref[...]
    o_ref[...] = 1.0 / (1.0 + jnp.exp(-v))


def kernel(x, edge_index, W1, a_src1, a_dst1, b1, W2, a_src2, a_dst2, b2):
    n = x.shape[0]
    f_in = x.shape[1]
    hc1 = _H1 * _C1

    loop = jnp.arange(n, dtype=edge_index.dtype)
    src = jnp.concatenate([edge_index[0], loop])
    dst = jnp.concatenate([edge_index[1], loop])
    e = src.shape[0]

    bn = 2048
    gn = _cdiv(n, bn)
    h, asrc, adst = pl.pallas_call(
        _node1_body,
        grid=(gn,),
        in_specs=[
            pl.BlockSpec((bn, f_in), lambda i: (i, 0)),
            pl.BlockSpec((f_in, hc1), lambda i: (0, 0)),
            pl.BlockSpec((_H1, _C1), lambda i: (0, 0)),
            pl.BlockSpec((_H1, _C1), lambda i: (0, 0)),
        ],
        out_specs=[
            pl.BlockSpec((bn, hc1), lambda i: (i, 0)),
            pl.BlockSpec((bn, _H1), lambda i: (i, 0)),
            pl.BlockSpec((bn, _H1), lambda i: (i, 0)),
        ],
        out_shape=[
            jax.ShapeDtypeStruct((n, hc1), jnp.float32),
            jax.ShapeDtypeStruct((n, _H1), jnp.float32),
            jax.ShapeDtypeStruct((n, _H1), jnp.float32),
        ],
    )(x, W1, a_src1, a_dst1)

    ae_s = jnp.take(asrc, src, axis=0)
    ae_d = jnp.take(adst, dst, axis=0)
    hs = jnp.take(h, src, axis=0)

    be = 2048
    ge = _cdiv(e, be)
    p1, msg = pl.pallas_call(
        _edge1_body,
        grid=(ge,),
        in_specs=[
            pl.BlockSpec((be, _H1), lambda i: (i, 0)),
            pl.BlockSpec((be, _H1), lambda i: (i, 0)),
            pl.BlockSpec((be, hc1), lambda i: (i, 0)),
        ],
        out_specs=[
            pl.BlockSpec((be, _H1), lambda i: (i, 0)),
            pl.BlockSpec((be, hc1), lambda i: (i, 0)),
        ],
        out_shape=[
            jax.ShapeDtypeStruct((e, _H1), jnp.float32),
            jax.ShapeDtypeStruct((e, hc1), jnp.float32),
        ],
    )(ae_s, ae_d, hs)

    den1 = jax.ops.segment_sum(p1, dst, num_segments=n)
    out1 = jax.ops.segment_sum(msg, dst, num_segments=n)

    z, zs, zd = pl.pallas_call(
        _node2_body,
        grid=(gn,),
        in_specs=[
            pl.BlockSpec((bn, hc1), lambda i: (i, 0)),
            pl.BlockSpec((bn, _H1), lambda i: (i, 0)),
            pl.BlockSpec((1, hc1), lambda i: (0, 0)),
            pl.BlockSpec((hc1, 1), lambda i: (0, 0)),
            pl.BlockSpec((1, 1), lambda i: (0, 0)),
            pl.BlockSpec((1, 1), lambda i: (0, 0)),
        ],
        out_specs=[
            pl.BlockSpec((bn, 1), lambda i: (i, 0)),
            pl.BlockSpec((bn, 1), lambda i: (i, 0)),
            pl.BlockSpec((bn, 1), lambda i: (i, 0)),
        ],
        out_shape=[
            jax.ShapeDtypeStruct((n, 1), jnp.float32),
            jax.ShapeDtypeStruct((n, 1), jnp.float32),
            jax.ShapeDtypeStruct((n, 1), jnp.float32),
        ],
    )(out1, den1, b1.reshape(1, hc1), W2, a_src2.reshape(1, 1),
      a_dst2.reshape(1, 1))

    ae2 = jnp.take(zs, src, axis=0) + jnp.take(zd, dst, axis=0)
    zg = jnp.take(z, src, axis=0)

    p2, m2 = pl.pallas_call(
        _edge2_body,
        grid=(ge,),
        in_specs=[
            pl.BlockSpec((be, 1), lambda i: (i, 0)),
            pl.BlockSpec((be, 1), lambda i: (i, 0)),
        ],
        out_specs=[
            pl.BlockSpec((be, 1), lambda i: (i, 0)),
            pl.BlockSpec((be, 1), lambda i: (i, 0)),
        ],
        out_shape=[
            jax.ShapeDtypeStruct((e, 1), jnp.float32),
            jax.ShapeDtypeStruct((e, 1), jnp.float32),
        ],
    )(ae2, zg)

    den2 = jax.ops.segment_sum(p2, dst, num_segments=n)
    num2 = jax.ops.segment_sum(m2, dst, num_segments=n)

    out = pl.pallas_call(
        _final_body,
        grid=(gn,),
        in_specs=[
            pl.BlockSpec((bn, 1), lambda i: (i, 0)),
            pl.BlockSpec((bn, 1), lambda i: (i, 0)),
            pl.BlockSpec((1, 1), lambda i: (0, 0)),
        ],
        out_specs=pl.BlockSpec((bn, 1), lambda i: (i, 0)),
        out_shape=jax.ShapeDtypeStruct((n, 1), jnp.float32),
    )(num2, den2, b2.reshape(1, 1))

    return out


# speedup vs baseline: 4.9653x; 1.0384x over previous
import functools

import jax
import jax.numpy as jnp
from jax.experimental import pallas as pl

_H1 = 8
_C1 = 8


def _cdiv(a, b):
    return (a + b - 1) // b


def _node1_body(x_ref, w_ref, asrc_ref, adst_ref, h_ref, s_ref, d_ref):
    h = jnp.dot(x_ref[...], w_ref[...], preferred_element_type=jnp.float32)
    h_ref[...] = h
    for hh in range(_H1):
        blk = h[:, hh * _C1:(hh + 1) * _C1]
        s_ref[:, hh:hh + 1] = jnp.sum(blk * asrc_ref[hh:hh + 1, :], axis=-1,
                                      keepdims=True)
        d_ref[:, hh:hh + 1] = jnp.sum(blk * adst_ref[hh:hh + 1, :], axis=-1,
                                      keepdims=True)


def _edge1_body(ae_s_ref, ae_d_ref, hs_ref, pm_ref):
    a = ae_s_ref[...] + ae_d_ref[...]
    a = jnp.where(a >= 0, a, 0.2 * a)
    p = jnp.exp(a)
    pm_ref[:, :_H1] = p
    for hh in range(_H1):
        pm_ref[:, _H1 + hh * _C1:_H1 + (hh + 1) * _C1] = (
            hs_ref[:, hh * _C1:(hh + 1) * _C1] * p[:, hh:hh + 1])


def _node2_body(out1_ref, den_ref, b1_ref, w2_ref, as2_ref, ad2_ref,
                z_ref, zs_ref, zd_ref):
    den = den_ref[...]
    cols = []
    for hh in range(_H1):
        cols.append(out1_ref[:, hh * _C1:(hh + 1) * _C1] /
                    (den[:, hh:hh + 1] + 1e-16))
    h2 = jnp.concatenate(cols, axis=1) + b1_ref[...]
    h2 = jnp.where(h2 > 0, h2, jnp.exp(h2) - 1.0)
    z = jnp.dot(h2, w2_ref[...], preferred_element_type=jnp.float32)
    z_ref[...] = z
    zs_ref[...] = z * as2_ref[...]
    zd_ref[...] = z * ad2_ref[...]


def _edge2_body(ae_ref, zg_ref, pm_ref):
    a = ae_ref[...]
    a = jnp.where(a >= 0, a, 0.2 * a)
    p = jnp.exp(a)
    pm_ref[:, 0:1] = p
    pm_ref[:, 1:2] = p * zg_ref[...]


def _final_body(num_ref, den_ref, b2_ref, o_ref):
    v = num_ref[...] / (den_ref[...] + 1e-16) + b2_ref[...]
    o_ref[...] = 1.0 / (1.0 + jnp.exp(-v))


def kernel(x, edge_index, W1, a_src1, a_dst1, b1, W2, a_src2, a_dst2, b2):
    n = x.shape[0]
    f_in = x.shape[1]
    hc1 = _H1 * _C1

    loop = jnp.arange(n, dtype=edge_index.dtype)
    src = jnp.concatenate([edge_index[0], loop])
    dst = jnp.concatenate([edge_index[1], loop])
    e = src.shape[0]

    bn = 2048
    gn = _cdiv(n, bn)
    h, asrc, adst = pl.pallas_call(
        _node1_body,
        grid=(gn,),
        in_specs=[
            pl.BlockSpec((bn, f_in), lambda i: (i, 0)),
            pl.BlockSpec((f_in, hc1), lambda i: (0, 0)),
            pl.BlockSpec((_H1, _C1), lambda i: (0, 0)),
            pl.BlockSpec((_H1, _C1), lambda i: (0, 0)),
        ],
        out_specs=[
            pl.BlockSpec((bn, hc1), lambda i: (i, 0)),
            pl.BlockSpec((bn, _H1), lambda i: (i, 0)),
            pl.BlockSpec((bn, _H1), lambda i: (i, 0)),
        ],
        out_shape=[
            jax.ShapeDtypeStruct((n, hc1), jnp.float32),
            jax.ShapeDtypeStruct((n, _H1), jnp.float32),
            jax.ShapeDtypeStruct((n, _H1), jnp.float32),
        ],
    )(x, W1, a_src1, a_dst1)

    ae_s = jnp.take(asrc, src, axis=0)
    ae_d = jnp.take(adst, dst, axis=0)
    hs = jnp.take(h, src, axis=0)

    be = 2048
    ge = _cdiv(e, be)
    pm1 = pl.pallas_call(
        _edge1_body,
        grid=(ge,),
        in_specs=[
            pl.BlockSpec((be, _H1), lambda i: (i, 0)),
            pl.BlockSpec((be, _H1), lambda i: (i, 0)),
            pl.BlockSpec((be, hc1), lambda i: (i, 0)),
        ],
        out_specs=pl.BlockSpec((be, _H1 + hc1), lambda i: (i, 0)),
        out_shape=jax.ShapeDtypeStruct((e, _H1 + hc1), jnp.float32),
    )(ae_s, ae_d, hs)

    seg1 = jax.ops.segment_sum(pm1, dst, num_segments=n)
    den1 = seg1[:, :_H1]
    out1 = seg1[:, _H1:]

    z, zs, zd = pl.pallas_call(
        _node2_body,
        grid=(gn,),
        in_specs=[
            pl.BlockSpec((bn, hc1), lambda i: (i, 0)),
            pl.BlockSpec((bn, _H1), lambda i: (i, 0)),
            pl.BlockSpec((1, hc1), lambda i: (0, 0)),
            pl.BlockSpec((hc1, 1), lambda i: (0, 0)),
            pl.BlockSpec((1, 1), lambda i: (0, 0)),
            pl.BlockSpec((1, 1), lambda i: (0, 0)),
        ],
        out_specs=[
            pl.BlockSpec((bn, 1), lambda i: (i, 0)),
            pl.BlockSpec((bn, 1), lambda i: (i, 0)),
            pl.BlockSpec((bn, 1), lambda i: (i, 0)),
        ],
        out_shape=[
            jax.ShapeDtypeStruct((n, 1), jnp.float32),
            jax.ShapeDtypeStruct((n, 1), jnp.float32),
            jax.ShapeDtypeStruct((n, 1), jnp.float32),
        ],
    )(out1, den1, b1.reshape(1, hc1), W2, a_src2.reshape(1, 1),
      a_dst2.reshape(1, 1))

    ae2 = jnp.take(zs, src, axis=0) + jnp.take(zd, dst, axis=0)
    zg = jnp.take(z, src, axis=0)

    pm2 = pl.pallas_call(
        _edge2_body,
        grid=(ge,),
        in_specs=[
            pl.BlockSpec((be, 1), lambda i: (i, 0)),
            pl.BlockSpec((be, 1), lambda i: (i, 0)),
        ],
        out_specs=pl.BlockSpec((be, 2), lambda i: (i, 0)),
        out_shape=jax.ShapeDtypeStruct((e, 2), jnp.float32),
    )(ae2, zg)

    seg2 = jax.ops.segment_sum(pm2, dst, num_segments=n)
    den2 = seg2[:, 0:1]
    num2 = seg2[:, 1:2]

    out = pl.pallas_call(
        _final_body,
        grid=(gn,),
        in_specs=[
            pl.BlockSpec((bn, 1), lambda i: (i, 0)),
            pl.BlockSpec((bn, 1), lambda i: (i, 0)),
            pl.BlockSpec((1, 1), lambda i: (0, 0)),
        ],
        out_specs=pl.BlockSpec((bn, 1), lambda i: (i, 0)),
        out_shape=jax.ShapeDtypeStruct((n, 1), jnp.float32),
    )(num2, den2, b2.reshape(1, 1))

    return out
